# Initial kernel scaffold; baseline (speedup 1.0000x reference)
import jax
import jax.numpy as jnp
import numpy as np
from jax import lax
from jax.experimental import pallas as pl
from jax.experimental.pallas import tpu as pltpu
from jax.experimental.pallas import tpu_sc as plsc

EPS = 1e-8
N_P = 200000
N_L = 50000
M = 400000
B = 256
G1 = 4096
T = 8
N_TOR = 4
CLAMP_RATE = 0.8
CLAMP_MAX = 20.0 / 10.0
COOR_SCALE2 = 100.0

K = 2000
NB = N_P // K
BPAD = 288

NW = 32
SEG_W = G1 // NW
COLS = 10
ACC_ROWS = SEG_W * COLS
C = 512
GPC = C // 16
M_PAD = M + 2 * C + 16

_ACOS_C = (1.5707288, -0.2121144, 0.0742610, -0.0187293)


def _fast_sqrt(x):
    i = lax.bitcast_convert_type(x, jnp.int32)
    y = lax.bitcast_convert_type(
        jnp.int32(0x5F3759DF) - lax.shift_right_arithmetic(i, 1), jnp.float32)
    y = y * (1.5 - 0.5 * x * y * y)
    y = y * (1.5 - 0.5 * x * y * y)
    return x * y


def _acos_poly(x, sqrt_fn):
    ax = jnp.abs(x)
    p = _ACOS_C[3]
    for c in (_ACOS_C[2], _ACOS_C[1], _ACOS_C[0]):
        p = p * ax + c
    r = sqrt_fn(jnp.maximum(1.0 - ax, 0.0)) * p
    return jnp.where(x >= 0, r, np.pi - r)


def _lig_body(pred_hbm, true_hbm, lm_hbm, lnm_hbm, ids_hbm, offs_hbm, out_hbm,
              idxm_v, idxn_v, ids_v, pred_v, true_v, acc_v, outb_v, offs_v,
              sem1, sem2):
    wid = lax.axis_index("s") * 2 + lax.axis_index("c")
    lane = lax.iota(jnp.int32, 16)
    zero16 = jnp.zeros((16,), jnp.float32)
    ones16 = jnp.ones((16,), jnp.float32)

    def _zero(i, _):
        acc_v[pl.ds(i * 16, 16)] = zero16
        return 0

    lax.fori_loop(0, (16 * ACC_ROWS) // 16, _zero, 0)

    pltpu.sync_copy(offs_hbm, offs_v)
    o0 = offs_v[pl.ds(0, 16)]
    o1 = offs_v[pl.ds(16, 16)]
    o2 = offs_v[pl.ds(32, 16)]

    def _sel_off(k):
        v = jnp.where(k // 16 == 0, o0, jnp.where(k // 16 == 1, o1, o2))
        return jnp.sum(jnp.where(lane == (k % 16), v, 0))

    e0 = _sel_off(wid)
    e1 = _sel_off(wid + 1)
    s0 = wid * SEG_W
    e0a = (e0 // 16) * 16
    nch = (e1 - e0a + C - 1) // C

    lane_off = lane * ACC_ROWS
    fcmax = jnp.full((16,), CLAMP_MAX, jnp.float32)

    def _chunk(kk, _):
        g0 = e0a + kk * C
        pltpu.sync_copy(lm_hbm.at[pl.ds(g0, C)], idxm_v)
        pltpu.sync_copy(lnm_hbm.at[pl.ds(g0, C)], idxn_v)
        pltpu.sync_copy(ids_hbm.at[pl.ds(g0, C)], ids_v)
        cp1 = pltpu.async_copy(pred_hbm.at[idxm_v], pred_v, sem1)
        cp2 = pltpu.async_copy(true_hbm.at[idxn_v], true_v, sem2)
        cp1.wait()
        cp2.wait()

        def _grp(j, _):
            gb = j * 16
            segs = ids_v[pl.ds(gb, 16)]
            rows = gb + lane
            gidx = g0 + rows
            msk = jnp.logical_and(gidx >= e0, gidx < e1)
            segl = jnp.clip(segs - s0, 0, SEG_W - 1)
            arow = lane_off + segl * COLS
            tx = plsc.load_gather(true_v, [rows, jnp.full((16,), 0, jnp.int32)])
            ty = plsc.load_gather(true_v, [rows, jnp.full((16,), 1, jnp.int32)])
            tz = plsc.load_gather(true_v, [rows, jnp.full((16,), 2, jnp.int32)])
            ss_last = zero16
            for t in range(T):
                cx = jnp.full((16,), 3 * t, jnp.int32)
                px = plsc.load_gather(pred_v, [rows, cx])
                py = plsc.load_gather(pred_v, [rows, cx + 1])
                pz = plsc.load_gather(pred_v, [rows, cx + 2])
                dx = px - tx
                dy = py - ty
                dz = pz - tz
                ss = dx * dx + dy * dy + dz * dz
                d = _fast_sqrt(ss)
                dmix = jnp.minimum(d, fcmax) * CLAMP_RATE + d * (1.0 - CLAMP_RATE)
                plsc.addupdate_scatter(acc_v, [arow + t], dmix, mask=msk)
                if t == T - 1:
                    ss_last = ss
            plsc.addupdate_scatter(acc_v, [arow + 8], ss_last * COOR_SCALE2,
                                   mask=msk)
            plsc.addupdate_scatter(acc_v, [arow + 9], ones16, mask=msk)
            return 0

        lax.fori_loop(0, GPC, _grp, 0)
        return 0

    lax.fori_loop(0, nch, _chunk, 0)

    def _red(i, _):
        s = acc_v[pl.ds(i * 16, 16)]
        for l in range(1, 16):
            s = s + acc_v[pl.ds(l * ACC_ROWS + i * 16, 16)]
        outb_v[pl.ds(i * 16, 16)] = s
        return 0

    lax.fori_loop(0, ACC_ROWS // 16, _red, 0)
    pltpu.sync_copy(outb_v, out_hbm.at[pl.ds(wid * ACC_ROWS, ACC_ROWS)])


def _lig_call(predp, truep, lmp, lnmp, idsp, offs):
    mesh = plsc.VectorSubcoreMesh(core_axis_name="c", subcore_axis_name="s")
    kfn = pl.kernel(
        _lig_body,
        mesh=mesh,
        out_type=jax.ShapeDtypeStruct((G1 * COLS,), jnp.float32),
        scratch_types=[
            pltpu.VMEM((C,), jnp.int32),
            pltpu.VMEM((C,), jnp.int32),
            pltpu.VMEM((C,), jnp.int32),
            pltpu.VMEM((C, 32), jnp.float32),
            pltpu.VMEM((C, 16), jnp.float32),
            pltpu.VMEM((16 * ACC_ROWS,), jnp.float32),
            pltpu.VMEM((ACC_ROWS,), jnp.float32),
            pltpu.VMEM((48,), jnp.int32),
            pltpu.SemaphoreType.DMA,
            pltpu.SemaphoreType.DMA,
        ],
    )
    return kfn(predp, truep, lmp, lnmp, idsp, offs)


def _dotT(a, b):
    return lax.dot_general(a, b, (((1,), (1,)), ((), ())),
                           preferred_element_type=jnp.float32)


def _pocket_kernel(ca_ref, pct_ref, cam_ref, cb_ref, cbt_ref, cbm_ref,
                   scp_ref, tv_ref, tva_ref, tm_ref, ids_ref, out_ref):
    i = pl.program_id(0)

    @pl.when(i == 0)
    def _():
        out_ref[...] = jnp.zeros((BPAD, 16), jnp.float32)

    f32 = jnp.float32
    i32 = jnp.int32

    r8 = lax.broadcasted_iota(i32, (8, 24), 0)
    c24 = lax.broadcasted_iota(i32, (8, 24), 1)
    sel_x = (c24 == 3 * r8).astype(f32)
    sel_y = (c24 == 3 * r8 + 1).astype(f32)
    sel_z = (c24 == 3 * r8 + 2).astype(f32)
    r4 = lax.broadcasted_iota(i32, (4, 8), 0)
    c8 = lax.broadcasted_iota(i32, (4, 8), 1)
    sel_a = (c8 == 2 * r4).astype(f32)
    sel_b = (c8 == 2 * r4 + 1).astype(f32)
    c3 = lax.broadcasted_iota(i32, (1, 3), 1)
    e1_3 = (c3 >= 0).astype(f32)
    e1_4 = (lax.broadcasted_iota(i32, (1, 4), 1) >= 0).astype(f32)
    id4 = (lax.broadcasted_iota(i32, (4, 4), 0)
           == lax.broadcasted_iota(i32, (4, 4), 1)).astype(f32)
    px_sel = (c3 == 0).astype(f32)
    py_sel = (c3 == 1).astype(f32)
    pz_sel = (c3 == 2).astype(f32)

    ca = ca_ref[...]
    pct = pct_ref[...]
    cax = _dotT(sel_x, ca)
    cay = _dotT(sel_y, ca)
    caz = _dotT(sel_z, ca)
    px = _dotT(px_sel, pct)
    py = _dotT(py_sel, pct)
    pz = _dotT(pz_sel, pct)
    camT = _dotT(jnp.ones((1, 1), f32), cam_ref[...])
    dx = cax - px
    dy = cay - py
    dz = caz - pz
    ca_cols = (dx * dx + dy * dy + dz * dz) * camT

    cbd = cb_ref[...] - cbt_ref[...]
    cbmT = _dotT(jnp.ones((1, 1), f32), cbm_ref[...])
    cb_col = _dotT(e1_3, cbd * cbd) * cbmT

    scp = scp_ref[...]
    a0 = _dotT(sel_a, scp)
    a1 = _dotT(sel_b, scp)
    b0 = _dotT(sel_a, tv_ref[...])
    b1 = _dotT(sel_b, tv_ref[...])
    g0 = _dotT(sel_a, tva_ref[...])
    g1 = _dotT(sel_b, tva_ref[...])
    tmT = _dotT(id4, tm_ref[...])

    d0 = a0 - b0
    d1 = a1 - b1
    l2 = d0 * d0 + d1 * d1
    h0 = a0 - g0
    h1 = a1 - g1
    l2a = h0 * h0 + h1 * h1
    l2m = jnp.minimum(l2, l2a)
    n_a = a0 * a0 + a1 * a1
    pel = jnp.abs(jnp.sqrt(n_a) - 1.0)
    scv = (l2m + 0.01 * pel) * tmT
    tmsum = _dotT(e1_4, tmT)
    sc_col = _dotT(e1_4, scv) / (tmsum + EPS)

    bc0 = jnp.concatenate([b0, g0], axis=0)
    bc1 = jnp.concatenate([b1, g1], axis=0)
    a0t = jnp.concatenate([a0, a0], axis=0)
    a1t = jnp.concatenate([a1, a1], axis=0)
    nat = jnp.concatenate([n_a, n_a], axis=0)
    dots = a0t * bc0 + a1t * bc1
    nb = bc0 * bc0 + bc1 * bc1
    cos = dots / (jnp.sqrt(nat * nb) + EPS)
    cos = jnp.clip(cos, -1.0 + 1e-6, 1.0 - 1e-6)
    err = _acos_poly(cos, jnp.sqrt)
    errm = jnp.minimum(err[0:4], err[4:8])
    tdd_col = _dotT(e1_4, errm * tmT) * (180.0 / np.pi) / (tmsum + EPS)

    vals = jnp.concatenate(
        [ca_cols, cb_col, sc_col, tdd_col, jnp.ones((1, K), f32),
         jnp.zeros((4, K), f32)], axis=0)

    ids = ids_ref[0]
    lo = ids_ref[0, 0, 0]
    hi = ids_ref[0, 0, K - 1]
    nwin = (hi - lo) // 32 + 1

    def _win(w, _):
        w0 = lo + w * 32
        oh = ((lax.broadcasted_iota(i32, (32, K), 0) + w0) == ids).astype(f32)
        part = _dotT(oh, vals)
        out_ref[pl.ds(w0, 32), :] = out_ref[pl.ds(w0, 32), :] + part
        return 0

    lax.fori_loop(0, nwin, _win, 0)


def _pocket_call(ca2, pct, cam, cb2, cbt, cbm, scp2, tv2, tva2, tm, ids3):
    row = lambda c: pl.BlockSpec((K, c), lambda i: (i, 0))
    return pl.pallas_call(
        _pocket_kernel,
        grid=(NB,),
        in_specs=[
            row(24), row(3), row(1), row(3), row(3), row(1),
            row(8), row(8), row(8), row(4),
            pl.BlockSpec((1, 1, K), lambda i: (i, 0, 0)),
        ],
        out_specs=pl.BlockSpec((BPAD, 16), lambda i: (0, 0)),
        out_shape=jax.ShapeDtypeStruct((BPAD, 16), jnp.float32),
    )(ca2, pct, cam, cb2, cbt, cbm, scp2, tv2, tva2, tm, ids3)


def _combine_kernel(pacc_ref, lacc_ref, affp_ref, afft_ref, affm_ref,
                    lenl_ref, ids2_ref, out_ref):
    f32 = jnp.float32
    i32 = jnp.int32
    pacc = pacc_ref[...][0:B]
    cnt = jnp.maximum(pacc[:, 11:12], 1.0)
    ca_mean = pacc[:, 0:8] / cnt
    ca_vec = jnp.sum(ca_mean[:, 0:7], axis=1, keepdims=True) / 7.0 \
        + ca_mean[:, 7:8]
    ca_loss = jnp.sum(ca_vec) / B
    cb_loss = jnp.sum(pacc[:, 8:9] / cnt) / B
    sc_loss = jnp.sum(pacc[:, 9:10] / cnt) / B
    tdd = jnp.sum(pacc[:, 10:11] / cnt) / B

    affd = affp_ref[...] - afft_ref[...]
    aff_loss = jnp.sum(affd * affd * affm_ref[...]) / B

    lacc = lacc_ref[...]
    lcnt = jnp.maximum(lacc[:, 9:10], 1.0)
    lmean = lacc[:, 0:8] / lcnt
    cl2 = lacc[:, 8:9]

    sel = ids2_ref[...] == lax.broadcasted_iota(i32, (1, B), 1)
    inf = jnp.float32(np.inf)

    mins = []
    for t in range(T):
        bigv = jnp.where(sel, lmean[:, t:t + 1], inf)
        mins.append(jnp.min(bigv, axis=0, keepdims=True))
    coor_min = jnp.concatenate(mins, axis=0)
    coor_vec = jnp.sum(coor_min[0:7], axis=0, keepdims=True) / 7.0 \
        + coor_min[7:8]
    coor_loss = jnp.sum(coor_vec) / B

    cl2_min = jnp.min(jnp.where(sel, cl2, inf), axis=0, keepdims=True)
    rmsd = jnp.sqrt(cl2_min / lenl_ref[...])
    rmsd_value = jnp.sum(rmsd) / B
    rmsd_rate = jnp.sum((rmsd < 2.0).astype(f32)) / B

    grad_loss = (coor_loss + ca_loss + 0.5 * cb_loss + 0.5 * sc_loss
                 + aff_loss)

    lane16 = lax.broadcasted_iota(i32, (1, 16), 1)
    outv = jnp.zeros((1, 16), f32)
    for idx, s in enumerate((grad_loss, ca_loss, cb_loss, aff_loss, sc_loss,
                             tdd, coor_loss, rmsd_value, rmsd_rate)):
        outv = outv + jnp.where(lane16 == idx, s, 0.0)
    out_ref[...] = outv


def _combine_call(pacc, lacc, affp, afft, affm, lenl, ids2c):
    return pl.pallas_call(
        _combine_kernel,
        out_shape=jax.ShapeDtypeStruct((1, 16), jnp.float32),
    )(pacc, lacc, affp, afft, affm, lenl, ids2c)


def kernel(CA_pred, CB_pred, SC_pred, aff_pred, l_coor_pred, p_coor_true,
           p_CA_mask, p_CB_coor_true, p_CB_mask, aff_true, aff_mask,
           p_tor_vec_true, p_tor_vec_alt_true, p_tor_mask, l_coor_true,
           len_ligand, scatter_pocket, l_match, l_nomatch,
           scatter_ligand_1, scatter_ligand_2):
    f32 = jnp.float32
    i32 = jnp.int32

    predp = jnp.concatenate(
        [l_coor_pred.reshape(N_L, 24), jnp.zeros((N_L, 8), f32)], axis=1)
    truep = jnp.concatenate(
        [l_coor_true, jnp.zeros((N_L, 13), f32)], axis=1)
    padi = jnp.zeros((M_PAD - M,), i32)
    lmp = jnp.concatenate([l_match.astype(i32), padi])
    lnmp = jnp.concatenate([l_nomatch.astype(i32), padi])
    idsp = jnp.concatenate([scatter_ligand_1.astype(i32), padi])
    offs = jnp.searchsorted(
        scatter_ligand_1, jnp.arange(0, G1 + SEG_W, SEG_W, dtype=i32)
    ).astype(i32)
    offs = jnp.concatenate([offs, jnp.zeros((48 - offs.shape[0],), i32)])
    lacc = _lig_call(predp, truep, lmp, lnmp, idsp, offs).reshape(G1, COLS)

    pacc = _pocket_call(
        CA_pred.reshape(N_P, 24), p_coor_true, p_CA_mask.reshape(N_P, 1),
        CB_pred, p_CB_coor_true, p_CB_mask.reshape(N_P, 1),
        SC_pred.reshape(N_P, 8), p_tor_vec_true.reshape(N_P, 8),
        p_tor_vec_alt_true.reshape(N_P, 8), p_tor_mask,
        scatter_pocket.astype(i32).reshape(NB, 1, K))

    out = _combine_call(
        pacc, lacc,
        aff_pred.reshape(1, B), aff_true.reshape(1, B),
        aff_mask.reshape(1, B), len_ligand.reshape(1, B),
        scatter_ligand_2.astype(i32).reshape(G1, 1))

    return (out[0, 0], out[0, 1], out[0, 2], out[0, 3], out[0, 4],
            out[0, 5], out[0, 6], out[0, 7], out[0, 8])


# speedup vs baseline: 5.3198x; 5.3198x over previous
import jax
import jax.numpy as jnp
import numpy as np
from jax import lax
from jax.experimental import pallas as pl
from jax.experimental.pallas import tpu as pltpu
from jax.experimental.pallas import tpu_sc as plsc

EPS = 1e-8
N_P = 200000
N_L = 50000
M = 400000
B = 256
G1 = 4096
T = 8
CLAMP_RATE = 0.8
CLAMP_MAX = 20.0 / 10.0
COOR_SCALE2 = 100.0

K = 2000
NB = N_P // K
BPAD = 288

NW = 32
EDGES_W = 12800
M_PAD = NW * EDGES_W
C = 256
NCH = EDGES_W // C
KL = 2000
NBL = M // KL
G1PAD = G1 + 32

_ACOS_C = (1.5707288, -0.2121144, 0.0742610, -0.0187293)


def _acos_poly(x):
    ax = jnp.abs(x)
    p = _ACOS_C[3]
    for c in (_ACOS_C[2], _ACOS_C[1], _ACOS_C[0]):
        p = p * ax + c
    r = jnp.sqrt(jnp.maximum(1.0 - ax, 0.0)) * p
    return jnp.where(x >= 0, r, np.pi - r)


def _lig_body(pred_hbm, true_hbm, lm_hbm, lnm_hbm, out_hbm,
              idxm_v, idxn_v, idxgm_v, idxgn_v, pred_v, true_v, outb_v,
              sem1, sem2):
    wid = lax.axis_index("s") * 2 + lax.axis_index("c")
    base = wid * EDGES_W

    def _chunk(kk, _):
        g0 = base + kk * C
        pltpu.sync_copy(lm_hbm.at[pl.ds(g0, C)], idxm_v)
        pltpu.sync_copy(lnm_hbm.at[pl.ds(g0, C)], idxn_v)

        def _shift(i, _):
            mv = idxm_v[pl.ds(i * 16, 16)]
            nv = idxn_v[pl.ds(i * 16, 16)]
            idxgm_v[pl.ds(i * 16, 16)] = lax.shift_right_logical(mv, 2)
            idxgn_v[pl.ds(i * 16, 16)] = lax.shift_right_logical(nv, 2)
            return 0

        lax.fori_loop(0, C // 16, _shift, 0)
        cp1 = pltpu.async_copy(pred_hbm.at[idxgm_v], pred_v, sem1)
        cp2 = pltpu.async_copy(true_hbm.at[idxgn_v], true_v, sem2)
        cp1.wait()
        cp2.wait()

        def _edge16(jj, _):
            j16 = jj * 16
            mv = idxm_v[pl.ds(j16, 16)]
            nv = idxn_v[pl.ds(j16, 16)]
            for k in range(16):
                j = j16 + k
                subm = (mv[k] & 3) * 32
                subn = (nv[k] & 3) * 32
                p0 = pred_v[j, pl.ds(subm, 16)]
                p1 = pred_v[j, pl.ds(subm + 16, 16)]
                t0 = true_v[j, pl.ds(subn, 16)]
                t1 = true_v[j, pl.ds(subn + 16, 16)]
                d0 = p0 - t0
                d1 = p1 - t1
                outb_v[pl.ds(j * 32, 16)] = d0 * d0
                outb_v[pl.ds(j * 32 + 16, 16)] = d1 * d1
            return 0

        lax.fori_loop(0, C // 16, _edge16, 0)
        pltpu.sync_copy(outb_v, out_hbm.at[pl.ds(g0 * 32, C * 32)])
        return 0

    lax.fori_loop(0, NCH, _chunk, 0)


def _lig_call(pred4, true4, lmp, lnmp):
    mesh = plsc.VectorSubcoreMesh(core_axis_name="c", subcore_axis_name="s")
    kfn = pl.kernel(
        _lig_body,
        mesh=mesh,
        out_type=jax.ShapeDtypeStruct((M_PAD * 32,), jnp.float32),
        scratch_types=[
            pltpu.VMEM((C,), jnp.int32),
            pltpu.VMEM((C,), jnp.int32),
            pltpu.VMEM((C,), jnp.int32),
            pltpu.VMEM((C,), jnp.int32),
            pltpu.VMEM((C, 128), jnp.float32),
            pltpu.VMEM((C, 128), jnp.float32),
            pltpu.VMEM((C * 32,), jnp.float32),
            pltpu.SemaphoreType.DMA,
            pltpu.SemaphoreType.DMA,
        ],
    )
    return kfn(pred4, true4, lmp, lnmp)


def _dotT(a, b):
    return lax.dot_general(a, b, (((1,), (1,)), ((), ())),
                           preferred_element_type=jnp.float32)


def _coor_kernel(d2_ref, ids_ref, out_ref):
    i = pl.program_id(0)

    @pl.when(i == 0)
    def _():
        out_ref[...] = jnp.zeros((G1PAD, 16), jnp.float32)

    f32 = jnp.float32
    i32 = jnp.int32
    r8 = lax.broadcasted_iota(i32, (8, 32), 0)
    c32 = lax.broadcasted_iota(i32, (8, 32), 1)
    red = jnp.logical_and(c32 >= 3 * r8, c32 < 3 * r8 + 3).astype(f32)

    d2 = d2_ref[...]
    ss = _dotT(red, d2)
    d = jnp.sqrt(ss)
    dmix = (jnp.minimum(d, CLAMP_MAX) * CLAMP_RATE + d * (1.0 - CLAMP_RATE))
    cl2 = ss[7:8] * COOR_SCALE2
    vals = jnp.concatenate(
        [dmix, cl2, jnp.ones((1, KL), f32), jnp.zeros((6, KL), f32)], axis=0)

    ids = ids_ref[0]
    lo = ids_ref[0, 0, 0]
    hi = ids_ref[0, 0, KL - 1]
    nwin = (hi - lo) // 32 + 1

    def _win(w, _):
        w0 = lo + w * 32
        oh = ((lax.broadcasted_iota(i32, (32, KL), 0) + w0) == ids).astype(f32)
        part = _dotT(oh, vals)
        out_ref[pl.ds(w0, 32), :] = out_ref[pl.ds(w0, 32), :] + part
        return 0

    lax.fori_loop(0, nwin, _win, 0)


def _coor_call(d2, ids3):
    return pl.pallas_call(
        _coor_kernel,
        grid=(NBL,),
        in_specs=[
            pl.BlockSpec((KL, 32), lambda i: (i, 0)),
            pl.BlockSpec((1, 1, KL), lambda i: (i, 0, 0)),
        ],
        out_specs=pl.BlockSpec((G1PAD, 16), lambda i: (0, 0)),
        out_shape=jax.ShapeDtypeStruct((G1PAD, 16), jnp.float32),
    )(d2, ids3)


def _pocket_kernel(ca_ref, pct_ref, cam_ref, cb_ref, cbt_ref, cbm_ref,
                   scp_ref, tv_ref, tva_ref, tm_ref, ids_ref, out_ref):
    i = pl.program_id(0)

    @pl.when(i == 0)
    def _():
        out_ref[...] = jnp.zeros((BPAD, 16), jnp.float32)

    f32 = jnp.float32
    i32 = jnp.int32

    r8 = lax.broadcasted_iota(i32, (8, 24), 0)
    c24 = lax.broadcasted_iota(i32, (8, 24), 1)
    sel_x = (c24 == 3 * r8).astype(f32)
    sel_y = (c24 == 3 * r8 + 1).astype(f32)
    sel_z = (c24 == 3 * r8 + 2).astype(f32)
    r4 = lax.broadcasted_iota(i32, (4, 8), 0)
    c8 = lax.broadcasted_iota(i32, (4, 8), 1)
    sel_a = (c8 == 2 * r4).astype(f32)
    sel_b = (c8 == 2 * r4 + 1).astype(f32)
    c3 = lax.broadcasted_iota(i32, (1, 3), 1)
    e1_3 = (c3 >= 0).astype(f32)
    id4 = (lax.broadcasted_iota(i32, (4, 4), 0)
           == lax.broadcasted_iota(i32, (4, 4), 1)).astype(f32)
    px_sel = (c3 == 0).astype(f32)
    py_sel = (c3 == 1).astype(f32)
    pz_sel = (c3 == 2).astype(f32)

    ca = ca_ref[...]
    pct = pct_ref[...]
    cax = _dotT(sel_x, ca)
    cay = _dotT(sel_y, ca)
    caz = _dotT(sel_z, ca)
    px = _dotT(px_sel, pct)
    py = _dotT(py_sel, pct)
    pz = _dotT(pz_sel, pct)
    camT = _dotT(jnp.ones((1, 1), f32), cam_ref[...])
    dx = cax - px
    dy = cay - py
    dz = caz - pz
    ca_cols = (dx * dx + dy * dy + dz * dz) * camT

    cbd = cb_ref[...] - cbt_ref[...]
    cbmT = _dotT(jnp.ones((1, 1), f32), cbm_ref[...])
    cb_col = _dotT(e1_3, cbd * cbd) * cbmT

    scp = scp_ref[...]
    a0 = _dotT(sel_a, scp)
    a1 = _dotT(sel_b, scp)
    b0 = _dotT(sel_a, tv_ref[...])
    b1 = _dotT(sel_b, tv_ref[...])
    g0 = _dotT(sel_a, tva_ref[...])
    g1 = _dotT(sel_b, tva_ref[...])
    tmT = _dotT(id4, tm_ref[...])

    d0 = a0 - b0
    d1 = a1 - b1
    l2 = d0 * d0 + d1 * d1
    h0 = a0 - g0
    h1 = a1 - g1
    l2a = h0 * h0 + h1 * h1
    l2m = jnp.minimum(l2, l2a)
    n_a = a0 * a0 + a1 * a1
    pel = jnp.abs(jnp.sqrt(n_a) - 1.0)
    scv = (l2m + 0.01 * pel) * tmT
    tmsum = jnp.sum(tmT, axis=0, keepdims=True)
    sc_col = jnp.sum(scv, axis=0, keepdims=True) / (tmsum + EPS)

    bc0 = jnp.concatenate([b0, g0], axis=0)
    bc1 = jnp.concatenate([b1, g1], axis=0)
    a0t = jnp.concatenate([a0, a0], axis=0)
    a1t = jnp.concatenate([a1, a1], axis=0)
    nat = jnp.concatenate([n_a, n_a], axis=0)
    dots = a0t * bc0 + a1t * bc1
    nb = bc0 * bc0 + bc1 * bc1
    cos = dots / (jnp.sqrt(nat * nb) + EPS)
    cos = jnp.clip(cos, -1.0 + 1e-6, 1.0 - 1e-6)
    err = _acos_poly(cos)
    errm = jnp.minimum(err[0:4], err[4:8])
    tdd_col = (jnp.sum(errm * tmT, axis=0, keepdims=True)
               * (180.0 / np.pi) / (tmsum + EPS))

    vals = jnp.concatenate(
        [ca_cols, cb_col, sc_col, tdd_col, jnp.ones((1, K), f32),
         jnp.zeros((4, K), f32)], axis=0)

    ids = ids_ref[0]
    lo = ids_ref[0, 0, 0]
    hi = ids_ref[0, 0, K - 1]
    nwin = (hi - lo) // 32 + 1

    def _win(w, _):
        w0 = lo + w * 32
        oh = ((lax.broadcasted_iota(i32, (32, K), 0) + w0) == ids).astype(f32)
        part = _dotT(oh, vals)
        out_ref[pl.ds(w0, 32), :] = out_ref[pl.ds(w0, 32), :] + part
        return 0

    lax.fori_loop(0, nwin, _win, 0)


def _pocket_call(ca2, pct, cam, cb2, cbt, cbm, scp2, tv2, tva2, tm, ids3):
    row = lambda c: pl.BlockSpec((K, c), lambda i: (i, 0))
    return pl.pallas_call(
        _pocket_kernel,
        grid=(NB,),
        in_specs=[
            row(24), row(3), row(1), row(3), row(3), row(1),
            row(8), row(8), row(8), row(4),
            pl.BlockSpec((1, 1, K), lambda i: (i, 0, 0)),
        ],
        out_specs=pl.BlockSpec((BPAD, 16), lambda i: (0, 0)),
        out_shape=jax.ShapeDtypeStruct((BPAD, 16), jnp.float32),
    )(ca2, pct, cam, cb2, cbt, cbm, scp2, tv2, tva2, tm, ids3)


def _combine_kernel(pacc_ref, lacc_ref, affp_ref, afft_ref, affm_ref,
                    lenl_ref, ids2_ref, out_ref):
    f32 = jnp.float32
    i32 = jnp.int32
    pacc = pacc_ref[...][0:B]
    cnt = jnp.maximum(pacc[:, 11:12], 1.0)
    ca_mean = pacc[:, 0:8] / cnt
    ca_vec = jnp.sum(ca_mean[:, 0:7], axis=1, keepdims=True) / 7.0 \
        + ca_mean[:, 7:8]
    ca_loss = jnp.sum(ca_vec) / B
    cb_loss = jnp.sum(pacc[:, 8:9] / cnt) / B
    sc_loss = jnp.sum(pacc[:, 9:10] / cnt) / B
    tdd = jnp.sum(pacc[:, 10:11] / cnt) / B

    affd = affp_ref[...] - afft_ref[...]
    aff_loss = jnp.sum(affd * affd * affm_ref[...]) / B

    lacc = lacc_ref[...][0:G1]
    lcnt = jnp.maximum(lacc[:, 9:10], 1.0)
    lmean = lacc[:, 0:8] / lcnt
    cl2 = lacc[:, 8:9]

    sel = ids2_ref[...] == lax.broadcasted_iota(i32, (1, B), 1)
    inf = jnp.float32(np.inf)

    mins = []
    for t in range(T):
        bigv = jnp.where(sel, lmean[:, t:t + 1], inf)
        mins.append(jnp.min(bigv, axis=0, keepdims=True))
    coor_min = jnp.concatenate(mins, axis=0)
    coor_vec = jnp.sum(coor_min[0:7], axis=0, keepdims=True) / 7.0 \
        + coor_min[7:8]
    coor_loss = jnp.sum(coor_vec) / B

    cl2_min = jnp.min(jnp.where(sel, cl2, inf), axis=0, keepdims=True)
    rmsd = jnp.sqrt(cl2_min / lenl_ref[...])
    rmsd_value = jnp.sum(rmsd) / B
    rmsd_rate = jnp.sum((rmsd < 2.0).astype(f32)) / B

    grad_loss = (coor_loss + ca_loss + 0.5 * cb_loss + 0.5 * sc_loss
                 + aff_loss)

    lane16 = lax.broadcasted_iota(i32, (1, 16), 1)
    outv = jnp.zeros((1, 16), f32)
    for idx, s in enumerate((grad_loss, ca_loss, cb_loss, aff_loss, sc_loss,
                             tdd, coor_loss, rmsd_value, rmsd_rate)):
        outv = outv + jnp.where(lane16 == idx, s, 0.0)
    out_ref[...] = outv


def _combine_call(pacc, lacc, affp, afft, affm, lenl, ids2c):
    return pl.pallas_call(
        _combine_kernel,
        out_shape=jax.ShapeDtypeStruct((1, 16), jnp.float32),
    )(pacc, lacc, affp, afft, affm, lenl, ids2c)


def kernel(CA_pred, CB_pred, SC_pred, aff_pred, l_coor_pred, p_coor_true,
           p_CA_mask, p_CB_coor_true, p_CB_mask, aff_true, aff_mask,
           p_tor_vec_true, p_tor_vec_alt_true, p_tor_mask, l_coor_true,
           len_ligand, scatter_pocket, l_match, l_nomatch,
           scatter_ligand_1, scatter_ligand_2):
    f32 = jnp.float32
    i32 = jnp.int32

    predp = jnp.concatenate(
        [l_coor_pred.reshape(N_L, 24), jnp.zeros((N_L, 8), f32)], axis=1)
    pred4 = predp.reshape(N_L // 4, 128)
    truet = jnp.concatenate(
        [jnp.tile(l_coor_true, (1, T)), jnp.zeros((N_L, 8), f32)], axis=1)
    true4 = truet.reshape(N_L // 4, 128)
    padi = jnp.zeros((M_PAD - M,), i32)
    lmp = jnp.concatenate([l_match.astype(i32), padi])
    lnmp = jnp.concatenate([l_nomatch.astype(i32), padi])
    d2flat = _lig_call(pred4, true4, lmp, lnmp)
    d2 = d2flat[: M * 32].reshape(M, 32)
    lacc = _coor_call(d2, scatter_ligand_1.astype(i32).reshape(NBL, 1, KL))

    pacc = _pocket_call(
        CA_pred.reshape(N_P, 24), p_coor_true, p_CA_mask.reshape(N_P, 1),
        CB_pred, p_CB_coor_true, p_CB_mask.reshape(N_P, 1),
        SC_pred.reshape(N_P, 8), p_tor_vec_true.reshape(N_P, 8),
        p_tor_vec_alt_true.reshape(N_P, 8), p_tor_mask,
        scatter_pocket.astype(i32).reshape(NB, 1, K))

    out = _combine_call(
        pacc, lacc,
        aff_pred.reshape(1, B), aff_true.reshape(1, B),
        aff_mask.reshape(1, B), len_ligand.reshape(1, B),
        scatter_ligand_2.astype(i32).reshape(G1, 1))

    return (out[0, 0], out[0, 1], out[0, 2], out[0, 3], out[0, 4],
            out[0, 5], out[0, 6], out[0, 7], out[0, 8])
